# Initial kernel scaffold; baseline (speedup 1.0000x reference)
import jax
import jax.numpy as jnp
from jax.experimental import pallas as pl


def kernel(x, weight, bias):
    raise NotImplementedError("write your pallas kernel here")


# speedup vs baseline: 11.5220x; 11.5220x over previous
import functools

import jax
import jax.numpy as jnp
from jax.experimental import pallas as pl
from jax.experimental.pallas import tpu as pltpu


def _round_up(x, m):
    return ((x + m - 1) // m) * m


def _conv_kernel(x_ref, w_ref, o_ref, rhs_ref, *, offsets, cin, m, kpad):
    kc = cin * len(offsets)
    for t, d in enumerate(offsets):
        rhs_ref[t * cin:(t + 1) * cin, :] = x_ref[0, :, d:d + m]
    rhs_ref[kc:kpad, :] = jnp.ones((kpad - kc, m), jnp.bfloat16)
    o_ref[0] = jax.lax.dot_general(
        w_ref[...], rhs_ref[...],
        dimension_numbers=(((1,), (0,)), ((), ())),
        preferred_element_type=jnp.float32)


def kernel(x, weight, bias):
    N, Cin, H, W = x.shape
    Cout, Cin2, Kh, Kw = weight.shape
    assert Cin == Cin2
    Ho, Wo = H - Kh + 1, W - Kw + 1
    M = Ho * W
    offsets = tuple(kh * W + kw for kh in range(Kh) for kw in range(Kw))
    Kc = Cin * Kh * Kw
    Kpad = _round_up(Kc + 1, 8)
    Lpad = _round_up(M + offsets[-1], 128)

    x_b = jnp.pad(x.reshape(N, Cin, H * W),
                  ((0, 0), (0, 0), (0, Lpad - H * W))).astype(jnp.bfloat16)
    w_mat = weight.transpose(0, 2, 3, 1).reshape(Cout, Kc)
    w_b = jnp.zeros((Cout, Kpad), jnp.bfloat16)
    w_b = w_b.at[:, :Kc].set(w_mat.astype(jnp.bfloat16))
    w_b = w_b.at[:, Kc].set(bias.astype(jnp.bfloat16))

    out = pl.pallas_call(
        functools.partial(_conv_kernel, offsets=offsets, cin=Cin, m=M,
                          kpad=Kpad),
        out_shape=jax.ShapeDtypeStruct((N, Cout, M), jnp.float32),
        grid=(N,),
        in_specs=[
            pl.BlockSpec((1, Cin, Lpad), lambda n: (n, 0, 0)),
            pl.BlockSpec((Cout, Kpad), lambda n: (0, 0)),
        ],
        out_specs=pl.BlockSpec((1, Cout, M), lambda n: (n, 0, 0)),
        scratch_shapes=[pltpu.VMEM((Kpad, M), jnp.bfloat16)],
        compiler_params=pltpu.CompilerParams(
            dimension_semantics=("parallel",),
        ),
    )(x_b, w_b)

    return out.reshape(N, Cout, Ho, W)[:, :, :, :Wo]


# speedup vs baseline: 13.7191x; 1.1907x over previous
import functools

import jax
import jax.numpy as jnp
from jax.experimental import pallas as pl
from jax.experimental.pallas import tpu as pltpu


def _round_up(x, m):
    return ((x + m - 1) // m) * m


def _conv_kernel(x_ref, w_ref, o_ref, xb_ref, rhs_ref, *,
                 offsets, cin, m, kpad, hw, ho, w, wo):
    kc = cin * len(offsets)
    xb_ref[...] = x_ref[0].astype(jnp.bfloat16)
    for t, d in enumerate(offsets):
        md = min(m, hw - d)
        rhs_ref[t * cin:(t + 1) * cin, :md] = xb_ref[:, d:d + md]
    rhs_ref[kc:kpad, :] = jnp.ones((kpad - kc, m), jnp.bfloat16)
    acc = jax.lax.dot_general(
        w_ref[...], rhs_ref[...],
        dimension_numbers=(((1,), (0,)), ((), ())),
        preferred_element_type=jnp.float32)
    for h in range(ho):
        o_ref[0, :, h * wo:(h + 1) * wo] = acc[:, h * w:h * w + wo]


def kernel(x, weight, bias):
    N, Cin, H, W = x.shape
    Cout, Cin2, Kh, Kw = weight.shape
    assert Cin == Cin2
    Ho, Wo = H - Kh + 1, W - Kw + 1
    M = Ho * W
    offsets = tuple(kh * W + kw for kh in range(Kh) for kw in range(Kw))
    Kc = Cin * Kh * Kw
    Kpad = _round_up(Kc + 1, 8)

    w_mat = weight.transpose(0, 2, 3, 1).reshape(Cout, Kc)
    w_b = jnp.zeros((Cout, Kpad), jnp.bfloat16)
    w_b = w_b.at[:, :Kc].set(w_mat.astype(jnp.bfloat16))
    w_b = w_b.at[:, Kc].set(bias.astype(jnp.bfloat16))

    out = pl.pallas_call(
        functools.partial(_conv_kernel, offsets=offsets, cin=Cin, m=M,
                          kpad=Kpad, hw=H * W, ho=Ho, w=W, wo=Wo),
        out_shape=jax.ShapeDtypeStruct((N, Cout, Ho * Wo), jnp.float32),
        grid=(N,),
        in_specs=[
            pl.BlockSpec((1, Cin, H * W), lambda n: (n, 0, 0)),
            pl.BlockSpec((Cout, Kpad), lambda n: (0, 0)),
        ],
        out_specs=pl.BlockSpec((1, Cout, Ho * Wo), lambda n: (n, 0, 0)),
        scratch_shapes=[
            pltpu.VMEM((Cin, H * W), jnp.bfloat16),
            pltpu.VMEM((Kpad, M), jnp.bfloat16),
        ],
        compiler_params=pltpu.CompilerParams(
            dimension_semantics=("parallel",),
        ),
    )(x.reshape(N, Cin, H * W), w_b)

    return out.reshape(N, Cout, Ho, Wo)
